# baseline (speedup 1.0000x reference)
import functools

import jax
import jax.numpy as jnp
from jax import lax
from jax.experimental import pallas as pl
from jax.experimental.pallas import tpu as pltpu
from jax.experimental.pallas import tpu_sc as plsc

_NC = 2
_NS = 16
_LANES = 16
_NW = _NC * _NS


def _make_sc_gather(B, C, HW, M):
    n_items = B * M
    n = n_items // _NW
    assert n % _LANES == 0 and n % 8 == 0

    @functools.partial(
        pl.kernel,
        mesh=plsc.VectorSubcoreMesh(core_axis_name="c", subcore_axis_name="s"),
        out_type=(
            jax.ShapeDtypeStruct((n_items,), jnp.float32),
            jax.ShapeDtypeStruct((n_items,), jnp.float32),
            jax.ShapeDtypeStruct((n_items,), jnp.float32),
            jax.ShapeDtypeStruct((n_items,), jnp.float32),
            jax.ShapeDtypeStruct((n_items,), jnp.float32),
        ),
        scratch_types=[
            pltpu.VMEM((n,), jnp.int32),
            pltpu.VMEM((n,), jnp.int32),
            pltpu.VMEM((n,), jnp.int32),
            pltpu.VMEM((n,), jnp.float32),
            pltpu.SemaphoreType.DMA,
        ],
    )
    def sc_gather(hm_hbm, reg_hbm, wh_hbm, ind_hbm, cat_hbm,
                  pos_o, r0_o, r1_o, w0_o, w1_o,
                  ind_v, cat_v, idx_v, val_v, sem):
        wid = lax.axis_index("s") * _NC + lax.axis_index("c")
        base = wid * n
        b = base // M
        pltpu.sync_copy(ind_hbm.at[pl.ds(base, n)], ind_v)
        pltpu.sync_copy(cat_hbm.at[pl.ds(base, n)], cat_v)

        for j in range(n // _LANES):
            sl = pl.ds(j * _LANES, _LANES)
            idx_v[sl] = (b * C + cat_v[sl]) * HW + ind_v[sl]
        pltpu.async_copy(hm_hbm.at[idx_v], val_v, sem).wait()
        pltpu.sync_copy(val_v, pos_o.at[pl.ds(base, n)])

        for src, o0, o1 in ((reg_hbm, r0_o, r1_o), (wh_hbm, w0_o, w1_o)):
            for j in range(n // _LANES):
                sl = pl.ds(j * _LANES, _LANES)
                idx_v[sl] = (b * 2) * HW + ind_v[sl]
            pltpu.async_copy(src.at[idx_v], val_v, sem).wait()
            pltpu.sync_copy(val_v, o0.at[pl.ds(base, n)])
            for j in range(n // _LANES):
                sl = pl.ds(j * _LANES, _LANES)
                idx_v[sl] = idx_v[sl] + HW
            pltpu.async_copy(src.at[idx_v], val_v, sem).wait()
            pltpu.sync_copy(val_v, o1.at[pl.ds(base, n)])

    return sc_gather


def _tc_body(x_ref, g_ref, pos_ref, m_ref, rp_ref, rt_ref, rm_ref,
             wp_ref, wt_ref, wm_ref, out_ref, acc_ref):
    i = pl.program_id(0)
    nsteps = pl.num_programs(0)

    x = x_ref[...]
    g = g_ref[...]
    s = jnp.clip(1.0 / (1.0 + jnp.exp(-x)), 1e-4, 1.0 - 1e-4)
    gt = 1.0 - g
    gt2 = gt * gt
    part = jnp.sum(jnp.log(1.0 - s) * (s * s) * (gt2 * gt2))

    @pl.when(i == 0)
    def _():
        acc_ref[0] = part

    @pl.when(i > 0)
    def _():
        acc_ref[0] = acc_ref[0] + part

    @pl.when(i == nsteps - 1)
    def _():
        neg_loss = acc_ref[0]
        p = jnp.clip(1.0 / (1.0 + jnp.exp(-pos_ref[...])), 1e-4, 1.0 - 1e-4)
        mf = m_ref[...]
        num_pos = jnp.sum(mf)
        one_m_p = 1.0 - p
        pos_loss = jnp.sum(jnp.log(p) * (one_m_p * one_m_p) * mf)
        hm_loss = jnp.where(
            num_pos == 0.0,
            -neg_loss,
            -(pos_loss + neg_loss) / jnp.maximum(num_pos, 1.0),
        )

        rm = rm_ref[...]
        reg_loss = (jnp.sum(jnp.abs(rp_ref[...] * rm - rt_ref[...] * rm))
                    / (jnp.sum(rm) + 1e-4))
        wm = wm_ref[...]
        wh_loss = (jnp.sum(jnp.abs(wp_ref[...] * wm - wt_ref[...] * wm))
                   / (jnp.sum(wm) + 1e-4))

        out_ref[0] = 1.0 * hm_loss + 1.0 * reg_loss + 0.1 * wh_loss
        out_ref[1] = hm_loss
        out_ref[2] = reg_loss
        out_ref[3] = wh_loss


def kernel(hm_out, hm_gt, reg_out, reg_target, reg_mask,
           wh_out, wh_target, wh_mask, mask, ind, cat):
    B, C, H, W = hm_out.shape
    M = ind.shape[1]
    HW = H * W

    sc_gather = _make_sc_gather(B, C, HW, M)
    pos, r0, r1, w0, w1 = sc_gather(
        hm_out.reshape(-1), reg_out.reshape(-1), wh_out.reshape(-1),
        ind.reshape(-1), cat.reshape(-1))

    pos2 = pos.reshape(B, M)
    rp = jnp.stack([r0.reshape(B, M), r1.reshape(B, M)])
    wp = jnp.stack([w0.reshape(B, M), w1.reshape(B, M)])
    rt = jnp.moveaxis(reg_target, 2, 0)
    rm = jnp.moveaxis(reg_mask, 2, 0)
    wt = jnp.moveaxis(wh_target, 2, 0)
    wm = jnp.moveaxis(wh_mask, 2, 0)

    cols = 16384
    nrows = B * C * H * W // cols
    x2d = hm_out.reshape(nrows, cols)
    g2d = hm_gt.reshape(nrows, cols)
    br = 128
    grid = nrows // br

    small = lambda shp: pl.BlockSpec(shp, lambda i, _s=len(shp): (0,) * _s)
    out = pl.pallas_call(
        _tc_body,
        grid=(grid,),
        in_specs=[
            pl.BlockSpec((br, cols), lambda i: (i, 0)),
            pl.BlockSpec((br, cols), lambda i: (i, 0)),
            small((B, M)),
            small((B, M)),
            small((2, B, M)),
            small((2, B, M)),
            small((2, B, M)),
            small((2, B, M)),
            small((2, B, M)),
            small((2, B, M)),
        ],
        out_specs=pl.BlockSpec(memory_space=pltpu.SMEM),
        out_shape=jax.ShapeDtypeStruct((4,), jnp.float32),
        scratch_shapes=[pltpu.SMEM((1,), jnp.float32)],
    )(x2d, g2d, pos2, mask, rp, rt, rm, wp, wt, wm)

    return (out[0].reshape(()), out[1].reshape(()),
            out[2].reshape(()), out[3].reshape(()))
